# Initial kernel scaffold; baseline (speedup 1.0000x reference)
import jax
import jax.numpy as jnp
from jax.experimental import pallas as pl


def kernel(tokens_a, cu_seqlens_a, tokens_b, cu_seqlens_b):
    raise NotImplementedError("write your pallas kernel here")


# speedup vs baseline: 332.3740x; 332.3740x over previous
import functools

import jax
import jax.numpy as jnp
from jax import lax
from jax.experimental import pallas as pl
from jax.experimental.pallas import tpu as pltpu
from jax.experimental.pallas import tpu_sc as plsc

SEQ = 512
B = 4096
TOT = 1048576
CLS_ID = 101
SEP_ID = 102
LIMIT = SEQ - 3
FLOOR_HALF = LIMIT // 2
CEIL_HALF = LIMIT - FLOOR_HALF

NC = 2
NS = 16
NW = NC * NS
RPW = B // NW
WIN = 520
PADF = 16
BUF = 1056


def _body(tok_a, cu_a, tok_b, cu_b, out_w, out_m, out_t,
          cua_v, cub_v, bufa0, bufa1, bufb0, bufb1, wrow, mrow, trow, sems):
    bufa = (bufa0, bufa1)
    bufb = (bufb0, bufb1)
    cid = lax.axis_index("c")
    sid = lax.axis_index("s")
    wid = sid * NC + cid
    r0 = pl.multiple_of(wid * RPW, 8)

    pltpu.sync_copy(cu_a.at[pl.ds(r0, RPW + 8)], cua_v.at[pl.ds(0, RPW + 8)])
    pltpu.sync_copy(cu_b.at[pl.ds(r0, RPW + 8)], cub_v.at[pl.ds(0, RPW + 8)])

    def fetch(i, slot):
        sa0 = cua_v[pl.ds(i, 16)][0]
        sb0 = cub_v[pl.ds(i, 16)][0]
        astart = pl.multiple_of(jnp.minimum(sa0 & ~7, TOT - WIN), 8)
        bstart = pl.multiple_of(jnp.minimum(sb0 & ~7, TOT - WIN), 8)
        pltpu.async_copy(tok_a.at[pl.ds(astart, WIN)],
                         bufa[slot].at[pl.ds(PADF, WIN)], sems.at[slot, 0])
        pltpu.async_copy(tok_b.at[pl.ds(bstart, WIN)],
                         bufb[slot].at[pl.ds(PADF, WIN)], sems.at[slot, 1])

    def compute(i, slot):
        vca = cua_v[pl.ds(i, 16)]
        vcb = cub_v[pl.ds(i, 16)]
        sa0 = vca[0]
        sa1 = vca[1]
        sb0 = vcb[0]
        sb1 = vcb[1]
        la = sa1 - sa0
        lb = sb1 - sb0
        qa = jnp.minimum(la, CEIL_HALF + jnp.maximum(FLOOR_HALF - lb, 0))
        qb = jnp.minimum(lb, FLOOR_HALF + jnp.maximum(CEIL_HALF - la, 0))
        c1 = 1 + qa
        c2 = 2 + qa + qb
        pad_a = sa0 - jnp.minimum(sa0 & ~7, TOT - WIN)
        pad_b = sb0 - jnp.minimum(sb0 & ~7, TOT - WIN)

        pltpu.make_async_copy(tok_a.at[pl.ds(0, WIN)],
                              bufa[slot].at[pl.ds(PADF, WIN)],
                              sems.at[slot, 0]).wait()
        pltpu.make_async_copy(tok_b.at[pl.ds(0, WIN)],
                              bufb[slot].at[pl.ds(PADF, WIN)],
                              sems.at[slot, 1]).wait()

        nb = c2 // 16 + 1

        @pl.loop(0, nb)
        def _(j):
            j16 = j * 16
            pos = lax.iota(jnp.int32, 16) + j16
            va = bufa[slot][pl.ds(pad_a + j16 + (PADF - 1), 16)]
            bb = jnp.maximum(pad_b + j16 + (PADF - 2) - qa, 0)
            vb = bufb[slot][pl.ds(bb, 16)]
            w = jnp.where(pos < c1, va,
                jnp.where(pos == c1, SEP_ID,
                jnp.where(pos < c2, vb,
                jnp.where(pos == c2, SEP_ID, 0))))
            w = jnp.where(pos == 0, CLS_ID, w)
            m = jnp.where(pos <= c2, 1, 0)
            t = jnp.where((pos > c1) & (pos <= c2), 1, 0)
            wrow[pl.ds(j16, 16)] = w
            mrow[pl.ds(j16, 16)] = m
            trow[pl.ds(j16, 16)] = t

        zeros = jnp.zeros((16,), jnp.int32)

        @pl.loop(nb, SEQ // 16)
        def _(j):
            j16 = j * 16
            wrow[pl.ds(j16, 16)] = zeros
            mrow[pl.ds(j16, 16)] = zeros
            trow[pl.ds(j16, 16)] = zeros

        ro = pl.multiple_of((r0 + i) * SEQ, 8)
        pltpu.sync_copy(wrow, out_w.at[pl.ds(ro, SEQ)])
        pltpu.sync_copy(mrow, out_m.at[pl.ds(ro, SEQ)])
        pltpu.sync_copy(trow, out_t.at[pl.ds(ro, SEQ)])

    fetch(0, 0)

    @pl.loop(0, RPW, step=2)
    def _(i):
        fetch(i + 1, 1)
        compute(i, 0)

        @pl.when(i + 2 < RPW)
        def _():
            fetch(i + 2, 0)

        compute(i + 1, 1)


def kernel(tokens_a, cu_seqlens_a, tokens_b, cu_seqlens_b):
    cu_a = jnp.pad(cu_seqlens_a.astype(jnp.int32), (0, 7))
    cu_b = jnp.pad(cu_seqlens_b.astype(jnp.int32), (0, 7))
    mesh = plsc.VectorSubcoreMesh(core_axis_name="c", subcore_axis_name="s")
    out = jax.ShapeDtypeStruct((B * SEQ,), jnp.int32)
    f = pl.kernel(
        _body,
        out_type=(out, out, out),
        mesh=mesh,
        scratch_types=[
            pltpu.VMEM((RPW + 16,), jnp.int32),
            pltpu.VMEM((RPW + 16,), jnp.int32),
            pltpu.VMEM((BUF,), jnp.int32),
            pltpu.VMEM((BUF,), jnp.int32),
            pltpu.VMEM((BUF,), jnp.int32),
            pltpu.VMEM((BUF,), jnp.int32),
            pltpu.VMEM((SEQ,), jnp.int32),
            pltpu.VMEM((SEQ,), jnp.int32),
            pltpu.VMEM((SEQ,), jnp.int32),
            pltpu.SemaphoreType.DMA((2, 2)),
        ],
    )
    w, m, t = f(tokens_a.astype(jnp.int32), cu_a, tokens_b.astype(jnp.int32), cu_b)
    return (w.reshape(B, SEQ), m.reshape(B, SEQ), t.reshape(B, SEQ))


# speedup vs baseline: 440.2502x; 1.3246x over previous
import jax
import jax.numpy as jnp
from jax import lax
from jax.experimental import pallas as pl
from jax.experimental.pallas import tpu as pltpu
from jax.experimental.pallas import tpu_sc as plsc

SEQ = 512
B = 4096
TOT = 1048576
CLS_ID = 101
SEP_ID = 102
LIMIT = SEQ - 3
FLOOR_HALF = LIMIT // 2
CEIL_HALF = LIMIT - FLOOR_HALF

NC = 2
NS = 16
NW = NC * NS
RPW = B // NW
WIN = 520
PADF = 16
BUF = 1056
NSLOT = 4
G = 8
GW = G * SEQ


def _body(tok_a, cu_a, tok_b, cu_b, out_w, out_m, out_t,
          cua_v, cub_v,
          ba0, ba1, ba2, ba3, bb0, bb1, bb2, bb3,
          w0, w1, m0, m1, t0, t1, semi, semo):
    bufa = (ba0, ba1, ba2, ba3)
    bufb = (bb0, bb1, bb2, bb3)
    wst = (w0, w1)
    mst = (m0, m1)
    tst = (t0, t1)

    cid = lax.axis_index("c")
    sid = lax.axis_index("s")
    wid = sid * NC + cid
    r0 = pl.multiple_of(wid * RPW, 8)

    pltpu.sync_copy(cu_a.at[pl.ds(r0, RPW + 8)], cua_v.at[pl.ds(0, RPW + 8)])
    pltpu.sync_copy(cu_b.at[pl.ds(r0, RPW + 8)], cub_v.at[pl.ds(0, RPW + 8)])

    def fetch(row, slot):
        sa0 = cua_v[pl.ds(row, 16)][0]
        sb0 = cub_v[pl.ds(row, 16)][0]
        astart = pl.multiple_of(jnp.minimum(sa0 & ~7, TOT - WIN), 8)
        bstart = pl.multiple_of(jnp.minimum(sb0 & ~7, TOT - WIN), 8)
        pltpu.async_copy(tok_a.at[pl.ds(astart, WIN)],
                         bufa[slot].at[pl.ds(PADF, WIN)], semi.at[slot, 0])
        pltpu.async_copy(tok_b.at[pl.ds(bstart, WIN)],
                         bufb[slot].at[pl.ds(PADF, WIN)], semi.at[slot, 1])

    def wait_in(slot):
        pltpu.make_async_copy(tok_a.at[pl.ds(0, WIN)],
                              bufa[slot].at[pl.ds(PADF, WIN)],
                              semi.at[slot, 0]).wait()
        pltpu.make_async_copy(tok_b.at[pl.ds(0, WIN)],
                              bufb[slot].at[pl.ds(PADF, WIN)],
                              semi.at[slot, 1]).wait()

    def compute(row, slot, set_, k):
        vca = cua_v[pl.ds(row, 16)]
        vcb = cub_v[pl.ds(row, 16)]
        sa0 = vca[0]
        sa1 = vca[1]
        sb0 = vcb[0]
        sb1 = vcb[1]
        la = sa1 - sa0
        lb = sb1 - sb0
        qa = jnp.minimum(la, CEIL_HALF + jnp.maximum(FLOOR_HALF - lb, 0))
        qb = jnp.minimum(lb, FLOOR_HALF + jnp.maximum(CEIL_HALF - la, 0))
        c1 = 1 + qa
        c2 = 2 + qa + qb
        pad_a = sa0 - jnp.minimum(sa0 & ~7, TOT - WIN)
        pad_b = sb0 - jnp.minimum(sb0 & ~7, TOT - WIN)
        wrow, mrow, trow = wst[set_], mst[set_], tst[set_]
        ko = k * SEQ

        nb = c2 // 16 + 1

        @pl.loop(0, nb)
        def _(j):
            j16 = j * 16
            pos = lax.iota(jnp.int32, 16) + j16
            va = bufa[slot][pl.ds(pad_a + j16 + (PADF - 1), 16)]
            bb = jnp.maximum(pad_b + j16 + (PADF - 2) - qa, 0)
            vb = bufb[slot][pl.ds(bb, 16)]
            w = jnp.where(pos < c1, va,
                jnp.where(pos == c1, SEP_ID,
                jnp.where(pos < c2, vb,
                jnp.where(pos == c2, SEP_ID, 0))))
            w = jnp.where(pos == 0, CLS_ID, w)
            m = jnp.where(pos <= c2, 1, 0)
            t = jnp.where((pos > c1) & (pos <= c2), 1, 0)
            wrow[pl.ds(ko + j16, 16)] = w
            mrow[pl.ds(ko + j16, 16)] = m
            trow[pl.ds(ko + j16, 16)] = t

        zeros = jnp.zeros((16,), jnp.int32)

        @pl.loop(nb, SEQ // 16)
        def _(j):
            j16 = j * 16
            wrow[pl.ds(ko + j16, 16)] = zeros
            mrow[pl.ds(ko + j16, 16)] = zeros
            trow[pl.ds(ko + j16, 16)] = zeros

    def flush(base, set_):
        ro = pl.multiple_of((r0 + base) * SEQ, 8)
        pltpu.async_copy(wst[set_], out_w.at[pl.ds(ro, GW)], semo.at[set_, 0])
        pltpu.async_copy(mst[set_], out_m.at[pl.ds(ro, GW)], semo.at[set_, 1])
        pltpu.async_copy(tst[set_], out_t.at[pl.ds(ro, GW)], semo.at[set_, 2])

    def wait_out(set_):
        pltpu.make_async_copy(wst[set_], out_w.at[pl.ds(0, GW)],
                              semo.at[set_, 0]).wait()
        pltpu.make_async_copy(mst[set_], out_m.at[pl.ds(0, GW)],
                              semo.at[set_, 1]).wait()
        pltpu.make_async_copy(tst[set_], out_t.at[pl.ds(0, GW)],
                              semo.at[set_, 2]).wait()

    for s in range(NSLOT):
        fetch(s, s)

    @pl.loop(0, RPW, step=2 * G)
    def _(i):
        for set_ in range(2):
            base = i + set_ * G

            @pl.when(base >= 2 * G)
            def _():
                wait_out(set_)

            for k in range(G):
                row = base + k
                slot = (set_ * G + k) % NSLOT
                wait_in(slot)
                compute(row, slot, set_, k)
                nxt = row + NSLOT

                @pl.when(nxt < RPW)
                def _():
                    fetch(nxt, slot)

            flush(base, set_)

    wait_out(0)
    wait_out(1)


def kernel(tokens_a, cu_seqlens_a, tokens_b, cu_seqlens_b):
    cu_a = jnp.pad(cu_seqlens_a.astype(jnp.int32), (0, 7))
    cu_b = jnp.pad(cu_seqlens_b.astype(jnp.int32), (0, 7))
    mesh = plsc.VectorSubcoreMesh(core_axis_name="c", subcore_axis_name="s")
    out = jax.ShapeDtypeStruct((B * SEQ,), jnp.int32)
    f = pl.kernel(
        _body,
        out_type=(out, out, out),
        mesh=mesh,
        scratch_types=(
            [pltpu.VMEM((RPW + 16,), jnp.int32)] * 2
            + [pltpu.VMEM((BUF,), jnp.int32)] * (2 * NSLOT)
            + [pltpu.VMEM((GW,), jnp.int32)] * 6
            + [pltpu.SemaphoreType.DMA((NSLOT, 2)),
               pltpu.SemaphoreType.DMA((2, 3))]
        ),
    )
    w, m, t = f(tokens_a.astype(jnp.int32), cu_a, tokens_b.astype(jnp.int32), cu_b)
    return (w.reshape(B, SEQ), m.reshape(B, SEQ), t.reshape(B, SEQ))


# speedup vs baseline: 548.4244x; 1.2457x over previous
import jax
import jax.numpy as jnp
from jax import lax
from jax.experimental import pallas as pl
from jax.experimental.pallas import tpu as pltpu
from jax.experimental.pallas import tpu_sc as plsc

SEQ = 512
B = 4096
TOT = 1048576
CLS_ID = 101
SEP_ID = 102
LIMIT = SEQ - 3
FLOOR_HALF = LIMIT // 2
CEIL_HALF = LIMIT - FLOOR_HALF

NC = 2
NS = 16
NW = NC * NS
RPW = B // NW
WIN = 520
PADF = 16
BUF = 1056
NSLOT = 4
G = 8
GW = G * SEQ
RBLK = 256


def _sc_body(tok_a, cu_a, tok_b, cu_b, out_w,
             cua_v, cub_v,
             ba0, ba1, ba2, ba3, bb0, bb1, bb2, bb3,
             w0, w1, semi, semo):
    bufa = (ba0, ba1, ba2, ba3)
    bufb = (bb0, bb1, bb2, bb3)
    wst = (w0, w1)

    cid = lax.axis_index("c")
    sid = lax.axis_index("s")
    wid = sid * NC + cid
    r0 = pl.multiple_of(wid * RPW, 8)

    pltpu.sync_copy(cu_a.at[pl.ds(r0, RPW + 8)], cua_v.at[pl.ds(0, RPW + 8)])
    pltpu.sync_copy(cu_b.at[pl.ds(r0, RPW + 8)], cub_v.at[pl.ds(0, RPW + 8)])

    def fetch(row, slot):
        sa0 = cua_v[pl.ds(row, 16)][0]
        sb0 = cub_v[pl.ds(row, 16)][0]
        astart = pl.multiple_of(jnp.minimum(sa0 & ~7, TOT - WIN), 8)
        bstart = pl.multiple_of(jnp.minimum(sb0 & ~7, TOT - WIN), 8)
        pltpu.async_copy(tok_a.at[pl.ds(astart, WIN)],
                         bufa[slot].at[pl.ds(PADF, WIN)], semi.at[slot, 0])
        pltpu.async_copy(tok_b.at[pl.ds(bstart, WIN)],
                         bufb[slot].at[pl.ds(PADF, WIN)], semi.at[slot, 1])

    def wait_in(slot):
        pltpu.make_async_copy(tok_a.at[pl.ds(0, WIN)],
                              bufa[slot].at[pl.ds(PADF, WIN)],
                              semi.at[slot, 0]).wait()
        pltpu.make_async_copy(tok_b.at[pl.ds(0, WIN)],
                              bufb[slot].at[pl.ds(PADF, WIN)],
                              semi.at[slot, 1]).wait()

    def compute(row, slot, set_, k):
        vca = cua_v[pl.ds(row, 16)]
        vcb = cub_v[pl.ds(row, 16)]
        sa0 = vca[0]
        sa1 = vca[1]
        sb0 = vcb[0]
        sb1 = vcb[1]
        la = sa1 - sa0
        lb = sb1 - sb0
        qa = jnp.minimum(la, CEIL_HALF + jnp.maximum(FLOOR_HALF - lb, 0))
        qb = jnp.minimum(lb, FLOOR_HALF + jnp.maximum(CEIL_HALF - la, 0))
        c1 = 1 + qa
        c2 = 2 + qa + qb
        pad_a = sa0 - jnp.minimum(sa0 & ~7, TOT - WIN)
        pad_b = sb0 - jnp.minimum(sb0 & ~7, TOT - WIN)
        wrow = wst[set_]
        ko = k * SEQ

        nb = c2 // 16 + 1

        @pl.loop(0, nb)
        def _(j):
            j16 = j * 16
            pos = lax.iota(jnp.int32, 16) + j16
            va = bufa[slot][pl.ds(pad_a + j16 + (PADF - 1), 16)]
            bb = jnp.maximum(pad_b + j16 + (PADF - 2) - qa, 0)
            vb = bufb[slot][pl.ds(bb, 16)]
            w = jnp.where(pos < c1, va,
                jnp.where(pos == c1, SEP_ID,
                jnp.where(pos < c2, vb,
                jnp.where(pos == c2, SEP_ID, 0))))
            w = jnp.where(pos == 0, CLS_ID, w)
            wrow[pl.ds(ko + j16, 16)] = w

        zeros = jnp.zeros((16,), jnp.int32)

        @pl.loop(nb, SEQ // 16)
        def _(j):
            wrow[pl.ds(ko + j * 16, 16)] = zeros

    def flush(base, set_):
        ro = pl.multiple_of((r0 + base) * SEQ, 8)
        pltpu.async_copy(wst[set_], out_w.at[pl.ds(ro, GW)], semo.at[set_])

    def wait_out(set_):
        pltpu.make_async_copy(wst[set_], out_w.at[pl.ds(0, GW)],
                              semo.at[set_]).wait()

    for s in range(NSLOT):
        fetch(s, s)

    @pl.loop(0, RPW, step=2 * G)
    def _(i):
        for set_ in range(2):
            base = i + set_ * G

            @pl.when(base >= 2 * G)
            def _():
                wait_out(set_)

            for k in range(G):
                row = base + k
                slot = (set_ * G + k) % NSLOT
                wait_in(slot)
                compute(row, slot, set_, k)
                nxt = row + NSLOT

                @pl.when(nxt < RPW)
                def _():
                    fetch(nxt, slot)

            flush(base, set_)

    wait_out(0)
    wait_out(1)


def _tc_body(la_ref, lb_ref, m_ref, t_ref):
    la = la_ref[...]
    lb = lb_ref[...]
    qa = jnp.minimum(la, CEIL_HALF + jnp.maximum(FLOOR_HALF - lb, 0))
    qb = jnp.minimum(lb, FLOOR_HALF + jnp.maximum(CEIL_HALF - la, 0))
    c1 = 1 + qa
    c2 = 2 + qa + qb
    pos = lax.broadcasted_iota(jnp.int32, (RBLK, SEQ), 1)
    m_ref[...] = jnp.where(pos <= c2, 1, 0)
    t_ref[...] = jnp.where((pos > c1) & (pos <= c2), 1, 0)


def kernel(tokens_a, cu_seqlens_a, tokens_b, cu_seqlens_b):
    cu_a32 = cu_seqlens_a.astype(jnp.int32)
    cu_b32 = cu_seqlens_b.astype(jnp.int32)
    cu_a = jnp.pad(cu_a32, (0, 7))
    cu_b = jnp.pad(cu_b32, (0, 7))
    mesh = plsc.VectorSubcoreMesh(core_axis_name="c", subcore_axis_name="s")
    out = jax.ShapeDtypeStruct((B * SEQ,), jnp.int32)
    sc = pl.kernel(
        _sc_body,
        out_type=out,
        mesh=mesh,
        scratch_types=(
            [pltpu.VMEM((RPW + 16,), jnp.int32)] * 2
            + [pltpu.VMEM((BUF,), jnp.int32)] * (2 * NSLOT)
            + [pltpu.VMEM((GW,), jnp.int32)] * 2
            + [pltpu.SemaphoreType.DMA((NSLOT, 2)),
               pltpu.SemaphoreType.DMA((2,))]
        ),
    )
    w = sc(tokens_a.astype(jnp.int32), cu_a, tokens_b.astype(jnp.int32), cu_b)

    la = (cu_a32[1:] - cu_a32[:-1]).reshape(B, 1)
    lb = (cu_b32[1:] - cu_b32[:-1]).reshape(B, 1)
    m, t = pl.pallas_call(
        _tc_body,
        out_shape=(jax.ShapeDtypeStruct((B, SEQ), jnp.int32),
                   jax.ShapeDtypeStruct((B, SEQ), jnp.int32)),
        grid=(B // RBLK,),
        in_specs=[pl.BlockSpec((RBLK, 1), lambda i: (i, 0)),
                  pl.BlockSpec((RBLK, 1), lambda i: (i, 0))],
        out_specs=(pl.BlockSpec((RBLK, SEQ), lambda i: (i, 0)),
                   pl.BlockSpec((RBLK, SEQ), lambda i: (i, 0))),
    )(la, lb)
    return (w.reshape(B, SEQ), m, t)
